# baseline (speedup 1.0000x reference)
import numpy as np

import jax
import jax.numpy as jnp
from jax import lax
from jax.experimental import pallas as pl
from jax.experimental.pallas import tpu as pltpu
from jax.experimental.pallas import tpu_sc as plsc

_BATCH = 16384
_D = 3200
_NUM_GROUPS = 10
_NUM_SLICES = 5
_CHUNK = 32
_GROUP_W = _NUM_SLICES * _CHUNK
_BLOCKS_PER_ROW = _D // _CHUNK

_info = plsc.get_sparse_core_info()
_NC = _info.num_cores
_NS = _info.num_subcores
_NW = _NC * _NS
_RPW = _BATCH // _NW
_GROWS = _RPW * _NUM_SLICES
_IDX_W = 128
_NSLAB = _GROWS // _IDX_W


def _build_indices():
    i = np.arange(_GROWS, dtype=np.int64)
    base = (i // _NUM_SLICES) * _BLOCKS_PER_ROW + (i % _NUM_SLICES) * _NUM_GROUPS
    w = np.arange(_NW, dtype=np.int64)[:, None, None]
    g = np.arange(_NUM_GROUPS, dtype=np.int64)[None, :, None]
    idx = w * (_RPW * _BLOCKS_PER_ROW) + g + base[None, None, :]
    return idx.astype(np.int32).reshape(_NW, _NUM_GROUPS, _NSLAB, _IDX_W)


_IDX_NP = _build_indices()


_NSPLIT = 4
_NBUF = 4
_PHROWS = _GROWS // _NSPLIT
_SLABS_PER_PHASE = _NSLAB // _NSPLIT


def _body(tbl, idx_hbm, *rest):
    outs = rest[:_NUM_GROUPS]
    idx_v = rest[_NUM_GROUPS]
    dst_v = rest[_NUM_GROUPS + 1 : _NUM_GROUPS + 1 + _NBUF]
    gsem = rest[_NUM_GROUPS + 1 + _NBUF]
    wsems = rest[_NUM_GROUPS + 2 + _NBUF :]
    wid = lax.axis_index("s") * _NC + lax.axis_index("c")
    row0 = wid * _RPW

    pltpu.make_async_copy(idx_hbm.at[wid], idx_v, gsem).start()
    pltpu.make_async_copy(idx_hbm.at[wid], idx_v, gsem).wait()

    def write_desc(p, b):
        g, s = divmod(p, _NSPLIT)
        return pltpu.make_async_copy(
            dst_v[b],
            outs[g].at[pl.ds(row0 * _NUM_SLICES + s * _PHROWS, _PHROWS)],
            wsems[b],
        )

    nphase = _NUM_GROUPS * _NSPLIT
    for p in range(nphase):
        g, s = divmod(p, _NSPLIT)
        b = p % _NBUF
        if p >= _NBUF:
            write_desc(p - _NBUF, b).wait()
        copies = []
        for k in range(_SLABS_PER_PHASE):
            copies.append(
                pltpu.make_async_copy(
                    tbl.at[idx_v.at[g, s * _SLABS_PER_PHASE + k]],
                    dst_v[b].at[pl.ds(k * _IDX_W, _IDX_W)],
                    gsem,
                )
            )
        for c in copies:
            c.start()
        for c in copies:
            c.wait()
        write_desc(p, b).start()
    for p in range(nphase - _NBUF, nphase):
        write_desc(p, p % _NBUF).wait()


def kernel(input_tensor):
    tbl = input_tensor.reshape(_BATCH * _BLOCKS_PER_ROW, _CHUNK)
    idx = jnp.asarray(_IDX_NP)
    out_type = [
        jax.ShapeDtypeStruct((_BATCH * _NUM_SLICES, _CHUNK), jnp.float32)
    ] * _NUM_GROUPS
    f = pl.kernel(
        _body,
        out_type=out_type,
        mesh=plsc.VectorSubcoreMesh(core_axis_name="c", subcore_axis_name="s"),
        scratch_types=(
            [pltpu.VMEM((_NUM_GROUPS, _NSLAB, _IDX_W), jnp.int32)]
            + [pltpu.VMEM((_PHROWS, _CHUNK), jnp.float32)] * _NBUF
            + [pltpu.SemaphoreType.DMA] * (1 + _NBUF)
        ),
        compiler_params=pltpu.CompilerParams(use_tc_tiling_on_sc=False),
    )
    outs = f(tbl, idx)
    return tuple(o.reshape(_BATCH, _GROUP_W) for o in outs)


# speedup vs baseline: 1.0392x; 1.0392x over previous
import jax
import jax.numpy as jnp
from jax import lax
from jax.experimental import pallas as pl
from jax.experimental.pallas import tpu as pltpu
from jax.experimental.pallas import tpu_sc as plsc

_BATCH = 16384
_D = 3200
_NUM_GROUPS = 10
_NUM_SLICES = 5
_CHUNK = 32
_GROUP_W = _NUM_SLICES * _CHUNK
_USED_D = _NUM_GROUPS * _NUM_SLICES * _CHUNK

_info = plsc.get_sparse_core_info()
_NC = _info.num_cores
_NS = _info.num_subcores
_NW = _NC * _NS
_RPW = _BATCH // _NW
_CR = 16
_NCHUNK = _RPW // _CR


def _body(in_hbm, *rest):
    outs = rest[:_NUM_GROUPS]
    in_bufs = rest[_NUM_GROUPS : _NUM_GROUPS + 2]
    out_bufs = rest[_NUM_GROUPS + 2 : _NUM_GROUPS + 4]
    rsems = rest[_NUM_GROUPS + 4 : _NUM_GROUPS + 6]
    wsems = rest[_NUM_GROUPS + 6 : _NUM_GROUPS + 8]
    wid = lax.axis_index("s") * _NC + lax.axis_index("c")
    row0 = wid * _RPW

    def read_desc(c, u):
        return pltpu.make_async_copy(
            in_hbm.at[pl.ds(row0 + c * _CR, _CR), pl.ds(0, _USED_D)],
            in_bufs[u],
            rsems[u],
        )

    def write_desc(c, u, g):
        return pltpu.make_async_copy(
            out_bufs[u].at[:, pl.ds(g * _GROUP_W, _GROUP_W)],
            outs[g].at[pl.ds(row0 + c * _CR, _CR), :],
            wsems[u],
        )

    read_desc(0, 0).start()
    read_desc(1, 1).start()

    def chunk_pair(c2, _):
        for u in (0, 1):
            c = c2 * 2 + u
            read_desc(c, u).wait()

            @pl.when(c >= 2)
            def _():
                for g in range(_NUM_GROUPS):
                    write_desc(c - 2, u, g).wait()

            def shuffle_row(r, _):
                for g in range(_NUM_GROUPS):
                    for j in range(_NUM_SLICES):
                        src = (j * _NUM_GROUPS + g) * _CHUNK
                        dst = g * _GROUP_W + j * _CHUNK
                        for k in (0, 16):
                            out_bufs[u][r, pl.ds(dst + k, 16)] = in_bufs[u][
                                r, pl.ds(src + k, 16)
                            ]
                return 0

            lax.fori_loop(0, _CR, shuffle_row, 0)

            for g in range(_NUM_GROUPS):
                write_desc(c, u, g).start()

            @pl.when(c + 2 < _NCHUNK)
            def _():
                read_desc(c + 2, u).start()

        return 0

    lax.fori_loop(0, _NCHUNK // 2, chunk_pair, 0)

    for u in (0, 1):
        for g in range(_NUM_GROUPS):
            write_desc(_NCHUNK - 2 + u, u, g).wait()


def kernel(input_tensor):
    out_type = [
        jax.ShapeDtypeStruct((_BATCH, _GROUP_W), jnp.float32)
    ] * _NUM_GROUPS
    f = pl.kernel(
        _body,
        out_type=out_type,
        mesh=plsc.VectorSubcoreMesh(core_axis_name="c", subcore_axis_name="s"),
        scratch_types=(
            [pltpu.VMEM((_CR, _USED_D), jnp.float32)] * 4
            + [pltpu.SemaphoreType.DMA] * 4
        ),
        compiler_params=pltpu.CompilerParams(use_tc_tiling_on_sc=False),
    )
    return tuple(f(input_tensor))


# speedup vs baseline: 2.0403x; 1.9634x over previous
import jax
import jax.numpy as jnp
from jax import lax
from jax.experimental import pallas as pl
from jax.experimental.pallas import tpu as pltpu
from jax.experimental.pallas import tpu_sc as plsc

_BATCH = 16384
_D = 3200
_NUM_GROUPS = 10
_NUM_SLICES = 5
_CHUNK = 32
_GROUP_W = _NUM_SLICES * _CHUNK
_READ_W = 1664

_info = plsc.get_sparse_core_info()
_NC = _info.num_cores
_NS = _info.num_subcores
_NW = _NC * _NS
_RPW = _BATCH // _NW
_CR = 8
_NCHUNK = _RPW // _CR


def _body(in_hbm, *rest):
    outs = rest[:_NUM_GROUPS]
    in_bufs = rest[_NUM_GROUPS : _NUM_GROUPS + 2]
    out_bufs = [
        rest[_NUM_GROUPS + 2 + u * _NUM_GROUPS :][:_NUM_GROUPS]
        for u in (0, 1)
    ]
    sems = rest[_NUM_GROUPS + 2 + 2 * _NUM_GROUPS :]
    rsems = sems[0:2]
    wsems = sems[2:4]
    wid = lax.axis_index("s") * _NC + lax.axis_index("c")
    row0 = wid * _RPW

    def read_desc(c, u):
        return pltpu.make_async_copy(
            in_hbm.at[pl.ds(row0 + c * _CR, _CR), pl.ds(0, _READ_W)],
            in_bufs[u],
            rsems[u],
        )

    def write_desc(c, u, g):
        return pltpu.make_async_copy(
            out_bufs[u][g],
            outs[g].at[pl.ds(row0 + c * _CR, _CR), :],
            wsems[u],
        )

    read_desc(0, 0).start()
    read_desc(1, 1).start()

    def chunk_pair(c2, _):
        for u in (0, 1):
            c = c2 * 2 + u
            read_desc(c, u).wait()

            @pl.when(c >= 2)
            def _():
                for g in range(_NUM_GROUPS):
                    write_desc(c - 2, u, g).wait()

            def shuffle_row(r, _):
                for g in range(_NUM_GROUPS):
                    for j in range(_NUM_SLICES):
                        src = (j * _NUM_GROUPS + g) * _CHUNK
                        dst = j * _CHUNK
                        for k in (0, 16):
                            out_bufs[u][g][r, pl.ds(dst + k, 16)] = in_bufs[
                                u
                            ][r, pl.ds(src + k, 16)]
                return 0

            lax.fori_loop(0, _CR, shuffle_row, 0)

            for g in range(_NUM_GROUPS):
                write_desc(c, u, g).start()

            @pl.when(c + 2 < _NCHUNK)
            def _():
                read_desc(c + 2, u).start()

        return 0

    lax.fori_loop(0, _NCHUNK // 2, chunk_pair, 0)

    for u in (0, 1):
        for g in range(_NUM_GROUPS):
            write_desc(_NCHUNK - 2 + u, u, g).wait()


def kernel(input_tensor):
    out_type = [
        jax.ShapeDtypeStruct((_BATCH, _GROUP_W), jnp.float32)
    ] * _NUM_GROUPS
    f = pl.kernel(
        _body,
        out_type=out_type,
        mesh=plsc.VectorSubcoreMesh(core_axis_name="c", subcore_axis_name="s"),
        scratch_types=(
            [pltpu.VMEM((_CR, _READ_W), jnp.float32)] * 2
            + [pltpu.VMEM((_CR, _GROUP_W), jnp.float32)] * (2 * _NUM_GROUPS)
            + [pltpu.SemaphoreType.DMA] * 4
        ),
        compiler_params=pltpu.CompilerParams(use_tc_tiling_on_sc=True),
    )
    return tuple(f(input_tensor))
